# baseline (speedup 1.0000x reference)
import functools

import jax
import jax.numpy as jnp
from jax import lax
from jax.experimental import pallas as pl
from jax.experimental.pallas import tpu as pltpu
from jax.experimental.pallas import tpu_sc as plsc

TB = 32
NW = 32
CHG = 16
CHS = 8


def _sc_permute_rows(src, idx3, D):
    B = src.shape[0]
    _, NCH, CH = idx3.shape
    mesh = plsc.VectorSubcoreMesh(core_axis_name="c", subcore_axis_name="s")

    @functools.partial(
        pl.kernel,
        mesh=mesh,
        out_type=jax.ShapeDtypeStruct((B, D), jnp.float32),
        scratch_types=[
            pltpu.VMEM((NCH, CH), jnp.int32),
            pltpu.VMEM((CH, D), jnp.float32),
            pltpu.VMEM((CH, D), jnp.float32),
            pltpu.SemaphoreType.DMA,
            pltpu.SemaphoreType.DMA,
            pltpu.SemaphoreType.DMA,
            pltpu.SemaphoreType.DMA,
        ],
    )
    def k(src_hbm, idx_hbm, dst_hbm, idx_v, buf0, buf1, r0, r1, w0, w1):
        wid = lax.axis_index("s") * 2 + lax.axis_index("c")
        base = wid * (NCH * CH)
        pltpu.sync_copy(idx_hbm.at[wid], idx_v)
        bufs = (buf0, buf1)
        rsems = (r0, r1)
        wsems = (w0, w1)
        hr = [None] * NCH
        hw = [None] * NCH
        hr[0] = pltpu.async_copy(src_hbm.at[idx_v.at[0]], bufs[0], rsems[0])
        for j in range(NCH):
            b = j % 2
            hr[j].wait()
            hw[j] = pltpu.async_copy(
                bufs[b], dst_hbm.at[pl.ds(base + j * CH, CH)], wsems[b]
            )
            if j + 1 < NCH:
                if j - 1 >= 0:
                    hw[j - 1].wait()
                bn = (j + 1) % 2
                hr[j + 1] = pltpu.async_copy(
                    src_hbm.at[idx_v.at[j + 1]], bufs[bn], rsems[bn]
                )
        if NCH >= 2:
            hw[NCH - 2].wait()
        hw[NCH - 1].wait()

    return k


def _tc_encode_kernel(starts_ref, x_ref, Ws1_ref, bs1_ref, Ws2_ref, bs2_ref,
                      out_ref, E, MAX_F):
    i = pl.program_id(0)
    t0 = i * TB
    rows = TB * MAX_F
    xb = x_ref[...]
    P = out_ref.shape[-1]
    rowid = lax.broadcasted_iota(jnp.int32, (rows, P), 0)
    for e in range(E):
        lo = jnp.clip(starts_ref[e] - t0, 0, TB)
        hi = jnp.clip(starts_ref[e + 1] - t0, 0, TB)

        @pl.when(hi > lo)
        def _():
            h = jnp.dot(xb, Ws1_ref[e], preferred_element_type=jnp.float32)
            h = h + bs1_ref[e][None, :]
            a = h / (1.0 + jnp.abs(h))
            o = jnp.dot(a, Ws2_ref[e], preferred_element_type=jnp.float32)
            o = o + bs2_ref[e][None, :]
            full = jnp.logical_and(lo == 0, hi == TB)

            @pl.when(full)
            def _():
                out_ref[...] = o

            @pl.when(jnp.logical_not(full))
            def _():
                mask = jnp.logical_and(rowid >= lo * MAX_F, rowid < hi * MAX_F)
                out_ref[...] = jnp.where(mask, o, out_ref[...])


@jax.jit
def kernel(x, Ws1, bs1, Ws2, bs2, eid):
    B, MAX_F, N = x.shape
    E, _, H = Ws1.shape
    P = Ws2.shape[-1]

    oh = (eid[:, None] == jnp.arange(E, dtype=eid.dtype)).astype(jnp.int32)
    counts = jnp.sum(oh, axis=0)
    starts = jnp.concatenate(
        [jnp.zeros((1,), jnp.int32), jnp.cumsum(counts).astype(jnp.int32)]
    )
    rank = jnp.sum(jnp.cumsum(oh, axis=0) * oh, axis=1) - 1
    pos = starts[eid] + rank
    order = jnp.zeros((B,), jnp.int32).at[pos].set(
        jnp.arange(B, dtype=jnp.int32)
    )

    x2 = x.reshape(B, MAX_F * N)
    order3 = order.reshape(NW, B // (NW * CHG), CHG)
    xs = _sc_permute_rows(x2, order3, MAX_F * N)(x2, order3)

    grid = B // TB
    grid_spec = pltpu.PrefetchScalarGridSpec(
        num_scalar_prefetch=1,
        grid=(grid,),
        in_specs=[
            pl.BlockSpec((TB * MAX_F, N), lambda i, s: (i, 0)),
            pl.BlockSpec((E, N, H), lambda i, s: (0, 0, 0)),
            pl.BlockSpec((E, H), lambda i, s: (0, 0)),
            pl.BlockSpec((E, H, P), lambda i, s: (0, 0, 0)),
            pl.BlockSpec((E, P), lambda i, s: (0, 0)),
        ],
        out_specs=pl.BlockSpec((TB * MAX_F, P), lambda i, s: (i, 0)),
    )
    outs = pl.pallas_call(
        functools.partial(_tc_encode_kernel, E=E, MAX_F=MAX_F),
        grid_spec=grid_spec,
        out_shape=jax.ShapeDtypeStruct((B * MAX_F, P), jnp.float32),
    )(starts, xs.reshape(B * MAX_F, N), Ws1, bs1, Ws2, bs2)

    outs2 = outs.reshape(B, MAX_F * P)
    pos3 = pos.reshape(NW, B // (NW * CHS), CHS)
    out = _sc_permute_rows(outs2, pos3, MAX_F * P)(outs2, pos3)
    return out.reshape(B, MAX_F, P)


# speedup vs baseline: 1.0213x; 1.0213x over previous
import functools

import jax
import jax.numpy as jnp
from jax import lax
from jax.experimental import pallas as pl
from jax.experimental.pallas import tpu as pltpu
from jax.experimental.pallas import tpu_sc as plsc

TB = 32
NW = 32
CHG = 16
CHS = 8


def _sc_permute_rows(src, idx3, D, scatter):
    B = src.shape[0]
    _, NCH, CH = idx3.shape
    mesh = plsc.VectorSubcoreMesh(core_axis_name="c", subcore_axis_name="s")

    @functools.partial(
        pl.kernel,
        mesh=mesh,
        out_type=jax.ShapeDtypeStruct((B, D), jnp.float32),
        scratch_types=[
            pltpu.VMEM((NCH, CH), jnp.int32),
            pltpu.VMEM((CH, D), jnp.float32),
            pltpu.VMEM((CH, D), jnp.float32),
            pltpu.SemaphoreType.DMA,
            pltpu.SemaphoreType.DMA,
            pltpu.SemaphoreType.DMA,
            pltpu.SemaphoreType.DMA,
        ],
    )
    def k(src_hbm, idx_hbm, dst_hbm, idx_v, buf0, buf1, r0, r1, w0, w1):
        wid = lax.axis_index("s") * 2 + lax.axis_index("c")
        base = wid * (NCH * CH)
        pltpu.sync_copy(idx_hbm.at[wid], idx_v)
        bufs = (buf0, buf1)
        rsems = (r0, r1)
        wsems = (w0, w1)

        def read_src(j, b):
            if scatter:
                return pltpu.async_copy(
                    src_hbm.at[pl.ds(base + j * CH, CH)], bufs[b], rsems[b]
                )
            return pltpu.async_copy(src_hbm.at[idx_v.at[j]], bufs[b], rsems[b])

        def write_dst(j, b):
            if scatter:
                return pltpu.async_copy(
                    bufs[b], dst_hbm.at[idx_v.at[j]], wsems[b]
                )
            return pltpu.async_copy(
                bufs[b], dst_hbm.at[pl.ds(base + j * CH, CH)], wsems[b]
            )

        hr = [None] * NCH
        hw = [None] * NCH
        hr[0] = read_src(0, 0)
        for j in range(NCH):
            b = j % 2
            hr[j].wait()
            hw[j] = write_dst(j, b)
            if j + 1 < NCH:
                if j - 1 >= 0:
                    hw[j - 1].wait()
                hr[j + 1] = read_src(j + 1, (j + 1) % 2)
        if NCH >= 2:
            hw[NCH - 2].wait()
        hw[NCH - 1].wait()

    return k


def _tc_encode_kernel(starts_ref, x_ref, Ws1_ref, bs1_ref, Ws2_ref, bs2_ref,
                      out_ref, E, MAX_F):
    i = pl.program_id(0)
    t0 = i * TB
    rows = TB * MAX_F
    xb = x_ref[...]
    P = out_ref.shape[-1]
    rowid = lax.broadcasted_iota(jnp.int32, (rows, P), 0)
    for e in range(E):
        lo = jnp.clip(starts_ref[e] - t0, 0, TB)
        hi = jnp.clip(starts_ref[e + 1] - t0, 0, TB)

        @pl.when(hi > lo)
        def _():
            h = jnp.dot(xb, Ws1_ref[e], preferred_element_type=jnp.float32)
            h = h + bs1_ref[e][None, :]
            a = h / (1.0 + jnp.abs(h))
            o = jnp.dot(a, Ws2_ref[e], preferred_element_type=jnp.float32)
            o = o + bs2_ref[e][None, :]
            full = jnp.logical_and(lo == 0, hi == TB)

            @pl.when(full)
            def _():
                out_ref[...] = o

            @pl.when(jnp.logical_not(full))
            def _():
                mask = jnp.logical_and(rowid >= lo * MAX_F, rowid < hi * MAX_F)
                out_ref[...] = jnp.where(mask, o, out_ref[...])


@jax.jit
def kernel(x, Ws1, bs1, Ws2, bs2, eid):
    B, MAX_F, N = x.shape
    E, _, H = Ws1.shape
    P = Ws2.shape[-1]

    oh = (eid[:, None] == jnp.arange(E, dtype=eid.dtype)).astype(jnp.int32)
    counts = jnp.sum(oh, axis=0)
    starts = jnp.concatenate(
        [jnp.zeros((1,), jnp.int32), jnp.cumsum(counts).astype(jnp.int32)]
    )
    rank = jnp.sum(jnp.cumsum(oh, axis=0) * oh, axis=1) - 1
    pos = jnp.sum(oh * starts[None, :E], axis=1) + rank

    x2 = x.reshape(B, MAX_F * N)
    posg = pos.reshape(NW, B // (NW * CHG), CHG)
    xs = _sc_permute_rows(x2, posg, MAX_F * N, scatter=True)(x2, posg)

    grid = B // TB
    grid_spec = pltpu.PrefetchScalarGridSpec(
        num_scalar_prefetch=1,
        grid=(grid,),
        in_specs=[
            pl.BlockSpec((TB * MAX_F, N), lambda i, s: (i, 0)),
            pl.BlockSpec((E, N, H), lambda i, s: (0, 0, 0)),
            pl.BlockSpec((E, H), lambda i, s: (0, 0)),
            pl.BlockSpec((E, H, P), lambda i, s: (0, 0, 0)),
            pl.BlockSpec((E, P), lambda i, s: (0, 0)),
        ],
        out_specs=pl.BlockSpec((TB * MAX_F, P), lambda i, s: (i, 0)),
    )
    outs = pl.pallas_call(
        functools.partial(_tc_encode_kernel, E=E, MAX_F=MAX_F),
        grid_spec=grid_spec,
        out_shape=jax.ShapeDtypeStruct((B * MAX_F, P), jnp.float32),
    )(starts, xs.reshape(B * MAX_F, N), Ws1, bs1, Ws2, bs2)

    outs2 = outs.reshape(B, MAX_F * P)
    poss = pos.reshape(NW, B // (NW * CHS), CHS)
    out = _sc_permute_rows(outs2, poss, MAX_F * P, scatter=False)(outs2, poss)
    return out.reshape(B, MAX_F, P)
